# initial kernel scaffold (unmeasured)
import jax
import jax.numpy as jnp
from jax import lax
from jax.experimental import pallas as pl
from jax.experimental.pallas import tpu as pltpu

N_DEV = 8

_sem_signal = getattr(pl, "semaphore_signal", None) or pltpu.semaphore_signal
_sem_wait = getattr(pl, "semaphore_wait", None) or pltpu.semaphore_wait
_DevId = getattr(pl, "DeviceIdType", None) or pltpu.DeviceIdType
_CompilerParams = getattr(pltpu, "CompilerParams", None) or getattr(
    pltpu, "TPUCompilerParams"
)


def kernel(A, B):
    M, K = A.shape
    N = B.shape[1]
    CH = M // N_DEV

    A16 = A.astype(jnp.bfloat16)
    B16 = B.astype(jnp.bfloat16)

    def body(a_ref, b_ref, out_ref, z_ref, comm_ref, send_sems, recv_sems,
             credit_sem):
        my = lax.axis_index("i")
        left = lax.rem(my + N_DEV - 1, N_DEV)
        right = lax.rem(my + 1, N_DEV)

        barrier = pltpu.get_barrier_semaphore()
        for nbr in (left, right):
            _sem_signal(barrier, inc=1, device_id=(nbr,),
                        device_id_type=_DevId.MESH)
        _sem_wait(barrier, 2)

        z_ref[...] = jnp.dot(a_ref[...], b_ref[...],
                             preferred_element_type=jnp.float32)

        def credit_wait(step):
            if step >= 2:
                _sem_wait(credit_sem, 1)

        def credit_signal(step):
            if step <= 11:
                _sem_signal(credit_sem, inc=1, device_id=(left,),
                            device_id_type=_DevId.MESH)

        for s in range(N_DEV - 1):
            slot = s % 2
            send_c = lax.rem(my - s + 2 * N_DEV, N_DEV)
            recv_c = lax.rem(my - s - 1 + 2 * N_DEV, N_DEV)
            credit_wait(s)
            rdma = pltpu.make_async_remote_copy(
                src_ref=z_ref.at[pl.ds(send_c * CH, CH), :],
                dst_ref=comm_ref.at[slot],
                send_sem=send_sems.at[slot],
                recv_sem=recv_sems.at[slot],
                device_id=(right,),
                device_id_type=_DevId.MESH,
            )
            rdma.start()
            rdma.wait()
            z_ref[pl.ds(recv_c * CH, CH), :] = (
                z_ref[pl.ds(recv_c * CH, CH), :] + comm_ref[slot]
            )
            credit_signal(s)

        own = lax.rem(my + 1, N_DEV)
        zc = z_ref[pl.ds(own * CH, CH), :]
        out_ref[pl.ds(own * CH, CH), :] = zc / (1.0 + jnp.exp(-zc))

        for t in range(N_DEV - 1):
            s = N_DEV - 1 + t
            slot = s % 2
            c = lax.rem(own - t + 2 * N_DEV, N_DEV)
            credit_wait(s)
            rdma = pltpu.make_async_remote_copy(
                src_ref=out_ref.at[pl.ds(c * CH, CH), :],
                dst_ref=out_ref.at[pl.ds(c * CH, CH), :],
                send_sem=send_sems.at[slot],
                recv_sem=recv_sems.at[slot],
                device_id=(right,),
                device_id_type=_DevId.MESH,
            )
            rdma.start()
            rdma.wait()
            credit_signal(s)

    return pl.pallas_call(
        body,
        out_shape=jax.ShapeDtypeStruct((M, N), jnp.float32),
        in_specs=[
            pl.BlockSpec(memory_space=pltpu.VMEM),
            pl.BlockSpec(memory_space=pltpu.VMEM),
        ],
        out_specs=pl.BlockSpec(memory_space=pltpu.VMEM),
        scratch_shapes=[
            pltpu.VMEM((M, N), jnp.float32),
            pltpu.VMEM((2, CH, N), jnp.float32),
            pltpu.SemaphoreType.DMA((2,)),
            pltpu.SemaphoreType.DMA((2,)),
            pltpu.SemaphoreType.REGULAR,
        ],
        compiler_params=_CompilerParams(collective_id=0),
    )(A16, B16)


# baseline (device time: 385833 ns/iter reference)
import jax
import jax.numpy as jnp
from jax import lax
from jax.experimental import pallas as pl
from jax.experimental.pallas import tpu as pltpu

N_DEV = 8

_sem_signal = getattr(pl, "semaphore_signal", None) or pltpu.semaphore_signal
_sem_wait = getattr(pl, "semaphore_wait", None) or pltpu.semaphore_wait
_DevId = getattr(pl, "DeviceIdType", None) or pltpu.DeviceIdType
_CompilerParams = getattr(pltpu, "CompilerParams", None) or getattr(
    pltpu, "TPUCompilerParams"
)


def kernel(A, B):
    M, K = A.shape
    N = B.shape[1]
    CH = M // N_DEV

    A16 = A.astype(jnp.bfloat16)
    B16 = B.astype(jnp.bfloat16)

    def body(a_ref, b_ref, out_ref, z_ref, comm_ref, send_sems, recv_sems,
             credit_sem):
        my = lax.axis_index("i")
        left = lax.rem(my + N_DEV - 1, N_DEV)
        right = lax.rem(my + 1, N_DEV)

        barrier = pltpu.get_barrier_semaphore()
        for nbr in (left, right):
            _sem_signal(barrier, inc=1, device_id=(nbr,),
                        device_id_type=_DevId.MESH)
        _sem_wait(barrier, 2)

        z_ref[...] = jnp.dot(a_ref[...], b_ref[...],
                             preferred_element_type=jnp.float32)

        def credit_wait(step):
            if step >= 2:
                _sem_wait(credit_sem, 1)

        def credit_signal(step):
            if step <= 11:
                _sem_signal(credit_sem, inc=1, device_id=(left,),
                            device_id_type=_DevId.MESH)

        for s in range(N_DEV - 1):
            slot = s % 2
            send_c = lax.rem(my - s + 2 * N_DEV, N_DEV)
            recv_c = lax.rem(my - s - 1 + 2 * N_DEV, N_DEV)
            credit_wait(s)
            rdma = pltpu.make_async_remote_copy(
                src_ref=z_ref.at[pl.ds(send_c * CH, CH), :],
                dst_ref=comm_ref.at[slot],
                send_sem=send_sems.at[slot],
                recv_sem=recv_sems.at[slot],
                device_id=(right,),
                device_id_type=_DevId.MESH,
            )
            rdma.start()
            rdma.wait()
            z_ref[pl.ds(recv_c * CH, CH), :] = (
                z_ref[pl.ds(recv_c * CH, CH), :] + comm_ref[slot]
            )
            credit_signal(s)

        own = lax.rem(my + 1, N_DEV)
        zc = z_ref[pl.ds(own * CH, CH), :]
        out_ref[pl.ds(own * CH, CH), :] = zc / (1.0 + jnp.exp(-zc))

        for t in range(N_DEV - 1):
            s = N_DEV - 1 + t
            slot = s % 2
            c = lax.rem(own - t + 2 * N_DEV, N_DEV)
            credit_wait(s)
            rdma = pltpu.make_async_remote_copy(
                src_ref=out_ref.at[pl.ds(c * CH, CH), :],
                dst_ref=out_ref.at[pl.ds(c * CH, CH), :],
                send_sem=send_sems.at[slot],
                recv_sem=recv_sems.at[slot],
                device_id=(right,),
                device_id_type=_DevId.MESH,
            )
            rdma.start()
            rdma.wait()
            credit_signal(s)

    return pl.pallas_call(
        body,
        out_shape=jax.ShapeDtypeStruct((M, N), jnp.float32),
        in_specs=[
            pl.BlockSpec(memory_space=pltpu.VMEM),
            pl.BlockSpec(memory_space=pltpu.VMEM),
        ],
        out_specs=pl.BlockSpec(memory_space=pltpu.VMEM),
        scratch_shapes=[
            pltpu.VMEM((M, N), jnp.float32),
            pltpu.VMEM((2, CH, N), jnp.float32),
            pltpu.SemaphoreType.DMA((2,)),
            pltpu.SemaphoreType.DMA((2,)),
            pltpu.SemaphoreType.REGULAR,
        ],
        compiler_params=_CompilerParams(
            collective_id=0,
            vmem_limit_bytes=60 * 1024 * 1024,
        ),
    )(A16, B16)


# device time: 148761 ns/iter; 2.5936x vs baseline; 2.5936x over previous
import jax
import jax.numpy as jnp
from jax import lax
from jax.experimental import pallas as pl
from jax.experimental.pallas import tpu as pltpu

N_DEV = 8

_sem_signal = getattr(pl, "semaphore_signal", None) or pltpu.semaphore_signal
_sem_wait = getattr(pl, "semaphore_wait", None) or pltpu.semaphore_wait
_DevId = getattr(pl, "DeviceIdType", None) or pltpu.DeviceIdType
_CompilerParams = getattr(pltpu, "CompilerParams", None) or getattr(
    pltpu, "TPUCompilerParams"
)


def kernel(A, B):
    M, K = A.shape
    N = B.shape[1]
    CH = M // (2 * N_DEV)
    HALF = M // 2

    A16 = A.astype(jnp.bfloat16)
    B16 = B.astype(jnp.bfloat16)

    def body(a_ref, b_ref, out_ref, z_ref, stage_cw, comm_cw, stage_ccw,
             comm_ccw, send_cw, recv_cw, send_ccw, recv_ccw, credit_cw,
             credit_ccw):
        my = lax.axis_index("i")
        left = lax.rem(my + N_DEV - 1, N_DEV)
        right = lax.rem(my + 1, N_DEV)

        barrier = pltpu.get_barrier_semaphore()
        for nbr in (left, right):
            _sem_signal(barrier, inc=1, device_id=(nbr,),
                        device_id_type=_DevId.MESH)
        _sem_wait(barrier, 2)

        z_ref[...] = jnp.dot(a_ref[...], b_ref[...],
                             preferred_element_type=jnp.float32)

        def credit_wait(step):
            if step >= 2:
                _sem_wait(credit_cw, 1)
                _sem_wait(credit_ccw, 1)

        def credit_signal(step):
            if step <= 11:
                _sem_signal(credit_cw, inc=1, device_id=(left,),
                            device_id_type=_DevId.MESH)
                _sem_signal(credit_ccw, inc=1, device_id=(right,),
                            device_id_type=_DevId.MESH)

        def ring_pair(cw_src, cw_dst, ccw_src, ccw_dst, slot):
            r_cw = pltpu.make_async_remote_copy(
                src_ref=cw_src, dst_ref=cw_dst,
                send_sem=send_cw.at[slot], recv_sem=recv_cw.at[slot],
                device_id=(right,), device_id_type=_DevId.MESH,
            )
            r_ccw = pltpu.make_async_remote_copy(
                src_ref=ccw_src, dst_ref=ccw_dst,
                send_sem=send_ccw.at[slot], recv_sem=recv_ccw.at[slot],
                device_id=(left,), device_id_type=_DevId.MESH,
            )
            r_cw.start()
            r_ccw.start()
            r_cw.wait()
            r_ccw.wait()

        for s in range(N_DEV - 1):
            slot = s % 2
            cw_send = lax.rem(my - s + 2 * N_DEV, N_DEV)
            cw_recv = lax.rem(my - s - 1 + 2 * N_DEV, N_DEV)
            ccw_send = lax.rem(my + s, N_DEV)
            ccw_recv = lax.rem(my + s + 1, N_DEV)
            credit_wait(s)
            stage_cw[slot] = z_ref[pl.ds(cw_send * CH, CH), :].astype(
                jnp.bfloat16)
            stage_ccw[slot] = z_ref[
                pl.ds(HALF + ccw_send * CH, CH), :].astype(jnp.bfloat16)
            ring_pair(stage_cw.at[slot], comm_cw.at[slot],
                      stage_ccw.at[slot], comm_ccw.at[slot], slot)
            z_ref[pl.ds(cw_recv * CH, CH), :] = (
                z_ref[pl.ds(cw_recv * CH, CH), :] + comm_cw[slot]
            )
            z_ref[pl.ds(HALF + ccw_recv * CH, CH), :] = (
                z_ref[pl.ds(HALF + ccw_recv * CH, CH), :] + comm_ccw[slot]
            )
            credit_signal(s)

        own_cw = lax.rem(my + 1, N_DEV)
        own_ccw = lax.rem(my + N_DEV - 1, N_DEV)
        zc = z_ref[pl.ds(own_cw * CH, CH), :]
        out_ref[pl.ds(own_cw * CH, CH), :] = (
            zc / (1.0 + jnp.exp(-zc))).astype(jnp.bfloat16)
        zc = z_ref[pl.ds(HALF + own_ccw * CH, CH), :]
        out_ref[pl.ds(HALF + own_ccw * CH, CH), :] = (
            zc / (1.0 + jnp.exp(-zc))).astype(jnp.bfloat16)

        for t in range(N_DEV - 1):
            s = N_DEV - 1 + t
            slot = s % 2
            cw_c = lax.rem(own_cw - t + 2 * N_DEV, N_DEV)
            ccw_c = lax.rem(own_ccw + t, N_DEV)
            credit_wait(s)
            ring_pair(
                out_ref.at[pl.ds(cw_c * CH, CH), :],
                out_ref.at[pl.ds(cw_c * CH, CH), :],
                out_ref.at[pl.ds(HALF + ccw_c * CH, CH), :],
                out_ref.at[pl.ds(HALF + ccw_c * CH, CH), :],
                slot,
            )
            credit_signal(s)

    return pl.pallas_call(
        body,
        out_shape=jax.ShapeDtypeStruct((M, N), jnp.bfloat16),
        in_specs=[
            pl.BlockSpec(memory_space=pltpu.VMEM),
            pl.BlockSpec(memory_space=pltpu.VMEM),
        ],
        out_specs=pl.BlockSpec(memory_space=pltpu.VMEM),
        scratch_shapes=[
            pltpu.VMEM((M, N), jnp.float32),
            pltpu.VMEM((2, CH, N), jnp.bfloat16),
            pltpu.VMEM((2, CH, N), jnp.bfloat16),
            pltpu.VMEM((2, CH, N), jnp.bfloat16),
            pltpu.VMEM((2, CH, N), jnp.bfloat16),
            pltpu.SemaphoreType.DMA((2,)),
            pltpu.SemaphoreType.DMA((2,)),
            pltpu.SemaphoreType.DMA((2,)),
            pltpu.SemaphoreType.DMA((2,)),
            pltpu.SemaphoreType.REGULAR,
            pltpu.SemaphoreType.REGULAR,
        ],
        compiler_params=_CompilerParams(
            collective_id=0,
            vmem_limit_bytes=60 * 1024 * 1024,
        ),
    )(A16, B16)


# device time: 108101 ns/iter; 3.5692x vs baseline; 1.3761x over previous
import jax
import jax.numpy as jnp
from jax import lax
from jax.experimental import pallas as pl
from jax.experimental.pallas import tpu as pltpu

N_DEV = 8
N_BF = 3
AXES = ((1, 2, 4), (2, 4, 1), (4, 1, 2))
BASES = (0, 640, 1280)
SIZES = (640, 640, 768)

_sem_signal = getattr(pl, "semaphore_signal", None) or pltpu.semaphore_signal
_sem_wait = getattr(pl, "semaphore_wait", None) or pltpu.semaphore_wait
_DevId = getattr(pl, "DeviceIdType", None) or pltpu.DeviceIdType
_CompilerParams = getattr(pltpu, "CompilerParams", None) or getattr(
    pltpu, "TPUCompilerParams"
)


def kernel(A, B):
    M, K = A.shape
    N = B.shape[1]

    A16 = A.astype(jnp.bfloat16)
    B16 = B.astype(jnp.bfloat16)

    def body(a_ref, b_ref, out_ref, z_ref,
             st0, st1, st2,
             c00, c10, c20, c01, c11, c21, c02, c12, c22,
             send_sems, recv_sems):
        stages = (st0, st1, st2)
        comms = ((c00, c01, c02), (c10, c11, c12), (c20, c21, c22))

        m = lax.axis_index("i")
        L = m ^ ((m >> 1) & 1)

        def partner(ab):
            pL = L ^ ab
            return pL ^ ((pL >> 1) & 1)

        barrier = pltpu.get_barrier_semaphore()
        for ab in (1, 2, 4):
            _sem_signal(barrier, inc=1, device_id=(partner(ab),),
                        device_id_type=_DevId.MESH)
        _sem_wait(barrier, 3)

        z_ref[...] = jnp.dot(a_ref[...], b_ref[...],
                             preferred_element_type=jnp.float32)

        offs = [jnp.int32(BASES[b]) for b in range(N_BF)]
        szs = [SIZES[b] for b in range(N_BF)]

        for k in range(3):
            started = []
            for b in range(N_BF):
                ab = AXES[b][k]
                half = szs[b] // 2
                keep_lower = (L & ab) == 0
                send_off = offs[b] + jnp.where(keep_lower, half, 0)
                keep_off = offs[b] + jnp.where(keep_lower, 0, half)
                stages[b][:half, :] = z_ref[
                    pl.ds(send_off, half), :].astype(jnp.bfloat16)
                rdma = pltpu.make_async_remote_copy(
                    src_ref=stages[b].at[:half, :],
                    dst_ref=comms[b][k],
                    send_sem=send_sems.at[b, k],
                    recv_sem=recv_sems.at[b, k],
                    device_id=(partner(ab),),
                    device_id_type=_DevId.MESH,
                )
                rdma.start()
                started.append((rdma, b, keep_off, half))
            for rdma, b, keep_off, half in started:
                rdma.wait()
                z_ref[pl.ds(keep_off, half), :] = (
                    z_ref[pl.ds(keep_off, half), :] + comms[b][k][...]
                )
                offs[b] = keep_off
                szs[b] = half

        for b in range(N_BF):
            zc = z_ref[pl.ds(offs[b], szs[b]), :]
            out_ref[pl.ds(offs[b], szs[b]), :] = (
                zc / (1.0 + jnp.exp(-zc))).astype(jnp.bfloat16)

        for j in range(3):
            k = 3 + j
            started = []
            for b in range(N_BF):
                ab = AXES[b][2 - j]
                rdma = pltpu.make_async_remote_copy(
                    src_ref=out_ref.at[pl.ds(offs[b], szs[b]), :],
                    dst_ref=out_ref.at[pl.ds(offs[b], szs[b]), :],
                    send_sem=send_sems.at[b, k],
                    recv_sem=recv_sems.at[b, k],
                    device_id=(partner(ab),),
                    device_id_type=_DevId.MESH,
                )
                rdma.start()
                started.append((rdma, b, ab))
            for rdma, b, ab in started:
                rdma.wait()
                keep_lower = (L & ab) == 0
                offs[b] = offs[b] - jnp.where(keep_lower, 0, szs[b])
                szs[b] = szs[b] * 2

    comm_shapes = [
        pltpu.VMEM((SIZES[b] >> (k + 1), N), jnp.bfloat16)
        for k in range(3) for b in range(N_BF)
    ]
    return pl.pallas_call(
        body,
        out_shape=jax.ShapeDtypeStruct((M, N), jnp.bfloat16),
        in_specs=[
            pl.BlockSpec(memory_space=pltpu.VMEM),
            pl.BlockSpec(memory_space=pltpu.VMEM),
        ],
        out_specs=pl.BlockSpec(memory_space=pltpu.VMEM),
        scratch_shapes=[
            pltpu.VMEM((M, N), jnp.float32),
            pltpu.VMEM((SIZES[0] // 2, N), jnp.bfloat16),
            pltpu.VMEM((SIZES[1] // 2, N), jnp.bfloat16),
            pltpu.VMEM((SIZES[2] // 2, N), jnp.bfloat16),
            *comm_shapes,
            pltpu.SemaphoreType.DMA((N_BF, 6)),
            pltpu.SemaphoreType.DMA((N_BF, 6)),
        ],
        compiler_params=_CompilerParams(
            collective_id=0,
            vmem_limit_bytes=60 * 1024 * 1024,
        ),
    )(A16, B16)
